# baseline (speedup 1.0000x reference)
import functools

import jax
import jax.numpy as jnp
from jax import lax
from jax.experimental import pallas as pl
from jax.experimental.pallas import tpu as pltpu
from jax.experimental.pallas import tpu_sc as plsc

_CHS = [3, 32, 64, 128, 256, 512, 1024]
_NS = [40962, 10242, 2562, 642, 162, 42]
_NPS = [41216, 10752, 2816, 704, 192, 48]
_BRS = [1792, 1536, 704, 176, 192, 48]
_NW = 32


def _sc_gather(table, idx2, out_shape):
    v, c = table.shape
    n_idx_rows, bs = idx2.shape
    s_slabs, _, wout = out_shape
    tpr = wout // c
    buf_mult = 2 if tpr > 1 else 1
    t_max = max(8, (230 * 1024 // (buf_mult * 4 * (wout + tpr))) // 8 * 8)
    flat = s_slabs * bs
    k = max(2, -(-flat // (_NW * t_max)))
    while True:
        nts = max(1, (_NW * k) // s_slabs)
        t = min(t_max, -(-(-(-bs // nts)) // 8) * 8)
        if -(-bs // t) * s_slabs <= _NW * k:
            break
        k += 1
    nts = -(-bs // t)
    nt = s_slabs * nts
    nslots = min(2, k)
    mesh = plsc.VectorSubcoreMesh(core_axis_name="c", subcore_axis_name="s")

    @functools.partial(
        pl.kernel,
        out_type=jax.ShapeDtypeStruct(out_shape, jnp.float32),
        mesh=mesh,
        scratch_types=[
            [pltpu.VMEM((tpr, t), jnp.int32) for _ in range(nslots)],
            [pltpu.VMEM((tpr * t, c), jnp.float32) for _ in range(nslots)],
            [pltpu.VMEM((t, wout), jnp.float32)
             for _ in range(nslots if tpr > 1 else 0)],
            [pltpu.SemaphoreType.DMA for _ in range(3 * nslots)],
        ],
        compiler_params=pltpu.CompilerParams(use_tc_tiling_on_sc=False),
    )
    def gk(table_hbm, idx_hbm, out_hbm, idx_v, rows_g, rows_w, sems):
        wid = lax.axis_index("s") * 2 + lax.axis_index("c")

        def pos(i):
            tile = jnp.minimum(wid + i * _NW, nt - 1)
            slab = tile // nts
            r0 = jnp.minimum((tile % nts) * t, bs - t)
            return slab, r0

        def start_idx(i, slot):
            slab, r0 = pos(i)
            return pltpu.async_copy(
                idx_hbm.at[pl.ds(slab * tpr, tpr), pl.ds(r0, t)],
                idx_v[slot], sems[slot],
            )

        def start_gathers(i, slot):
            hs = []
            for q in range(tpr):
                dst = (rows_g[slot] if tpr == 1
                       else rows_g[slot].at[pl.ds(q * t, t), :])
                hs.append(pltpu.async_copy(
                    table_hbm.at[idx_v[slot].at[q]], dst, sems[nslots + slot]))
            return hs

        def permute(slot):
            if tpr == 1:
                return

            def body(r, carry):
                for q in range(tpr):
                    for ch in range(c // 16):
                        rows_w[slot][r, pl.ds(q * c + ch * 16, 16)] = (
                            rows_g[slot][q * t + r, pl.ds(ch * 16, 16)])
                return carry

            lax.fori_loop(0, t, body, 0)

        def start_wb(i, slot):
            slab, r0 = pos(i)
            src = rows_g[slot] if tpr == 1 else rows_w[slot]
            return pltpu.async_copy(
                src, out_hbm.at[slab, pl.ds(r0, t)],
                sems[2 * nslots + slot],
            )

        idx_h = [None] * k
        g_h = [None] * k
        w_h = [None] * k
        for i in range(nslots):
            idx_h[i] = start_idx(i, i)
        for i in range(k):
            s = i % 2
            idx_h[i].wait()
            if i >= 2:
                w_h[i - 2].wait()
            g_h[i] = start_gathers(i, s)
            if i >= 1:
                for h in g_h[i - 1]:
                    h.wait()
                permute(1 - s)
                w_h[i - 1] = start_wb(i - 1, 1 - s)
                if 2 <= i + 1 < k:
                    idx_h[i + 1] = start_idx(i + 1, 1 - s)
        for h in g_h[k - 1]:
            h.wait()
        permute((k - 1) % 2)
        w_h[k - 1] = start_wb(k - 1, (k - 1) % 2)
        if k >= 2:
            w_h[k - 2].wait()
        w_h[k - 1].wait()

    return gk(table, idx2)


def _bn_lrelu(y, stats_sum, stats_sq, n, g, e):
    m = stats_sum * (1.0 / n)
    var = stats_sq * (1.0 / n) - m * m
    scale = g * lax.rsqrt(var + 1e-5)
    shift = e - m * scale
    yv = y * scale + shift
    return jnp.where(yv >= 0, yv, 0.1 * yv)


def _tc_conv(m7, w, bge, n, br):
    fused = m7.ndim == 2
    n_p = m7.shape[0] if fused else m7.shape[1]
    cout = w.shape[-1]
    nb = n_p // br

    nslab = None if fused else m7.shape[0]
    cw = None if fused else m7.shape[2]

    def matmul(m_ref, w_ref):
        if fused:
            return jnp.dot(m_ref[...], w_ref[...],
                           preferred_element_type=jnp.float32, precision=lax.Precision.HIGHEST)
        y = jnp.dot(m_ref[0], w_ref[pl.ds(0, cw), :],
                    preferred_element_type=jnp.float32, precision=lax.Precision.HIGHEST)
        for q in range(1, nslab):
            y += jnp.dot(m_ref[q], w_ref[pl.ds(q * cw, cw), :],
                         preferred_element_type=jnp.float32, precision=lax.Precision.HIGHEST)
        return y

    if nb == 1:
        def body1(m_ref, w_ref, bge_ref, out_ref):
            y = matmul(m_ref, w_ref) + bge_ref[0:1, :]
            rows = lax.broadcasted_iota(jnp.int32, (n_p, 1), 0)
            ym = jnp.where(rows < n, y, 0.0)
            s1 = jnp.sum(ym, axis=0, keepdims=True)
            s2 = jnp.sum(ym * ym, axis=0, keepdims=True)
            out_ref[...] = _bn_lrelu(y, s1, s2, n, bge_ref[1:2, :],
                                     bge_ref[2:3, :])

        return pl.pallas_call(
            body1,
            out_shape=jax.ShapeDtypeStruct((n_p, cout), jnp.float32),
        )(m7, w, bge)

    if fused:
        m_spec = pl.BlockSpec((br, m7.shape[1]), lambda p, j: ((1 - p) * j, 0))
        w_spec = pl.BlockSpec(w.shape, lambda p, j: (0, 0))
    else:
        m_spec = pl.BlockSpec((nslab, br, cw),
                              lambda p, j: (0, (1 - p) * j, 0))
        w_spec = pl.BlockSpec(w.shape, lambda p, j: (0, 0))

    def body(m_ref, w_ref, bge_ref, out_ref, yacc, stats):
        p = pl.program_id(0)
        j = pl.program_id(1)

        @pl.when(p == 0)
        def _():
            y = matmul(m_ref, w_ref) + bge_ref[0:1, :]
            yacc[pl.ds(j * br, br), :] = y
            rows = j * br + lax.broadcasted_iota(jnp.int32, (br, 1), 0)
            ym = jnp.where(rows < n, y, 0.0)
            s1 = jnp.sum(ym, axis=0, keepdims=True)
            s2 = jnp.sum(ym * ym, axis=0, keepdims=True)

            @pl.when(j == 0)
            def _():
                stats[0:1, :] = s1
                stats[1:2, :] = s2

            @pl.when(j > 0)
            def _():
                stats[0:1, :] += s1
                stats[1:2, :] += s2

        @pl.when(p == 1)
        def _():
            out_ref[...] = _bn_lrelu(
                yacc[pl.ds(j * br, br), :], stats[0:1, :], stats[1:2, :],
                n, bge_ref[1:2, :], bge_ref[2:3, :])

    return pl.pallas_call(
        body,
        grid=(2, nb),
        in_specs=[m_spec, w_spec,
                  pl.BlockSpec((3, cout), lambda p, j: (0, 0))],
        out_specs=pl.BlockSpec((br, cout), lambda p, j: (p * j, 0)),
        out_shape=jax.ShapeDtypeStruct((n_p, cout), jnp.float32),
        scratch_shapes=[
            pltpu.VMEM((n_p, cout), jnp.float32),
            pltpu.VMEM((2, cout), jnp.float32),
        ],
    )(m7, w, bge)


def _tc_poolmean(g7, br, pmat=None):
    nslab, nc_p, cw = g7.shape

    if pmat is None:
        def body2(g_ref, out_ref):
            out_ref[...] = jnp.sum(g_ref[...], axis=0) * (1.0 / 7.0)

        return pl.pallas_call(
            body2,
            grid=(nc_p // br,),
            in_specs=[pl.BlockSpec((nslab, br, cw), lambda j: (0, j, 0))],
            out_specs=pl.BlockSpec((br, cw), lambda j: (j, 0)),
            out_shape=jax.ShapeDtypeStruct((nc_p, cw), jnp.float32),
        )(g7)

    c = pmat.shape[1]

    def body3(g_ref, p_ref, out_ref):
        y = jnp.dot(g_ref[0], p_ref[pl.ds(0, cw), :],
                    preferred_element_type=jnp.float32, precision=lax.Precision.HIGHEST)
        for q in range(1, nslab):
            y += jnp.dot(g_ref[q], p_ref[pl.ds(q * cw, cw), :],
                         preferred_element_type=jnp.float32, precision=lax.Precision.HIGHEST)
        out_ref[...] = y

    return pl.pallas_call(
        body3,
        grid=(nc_p // br,),
        in_specs=[pl.BlockSpec((nslab, br, cw), lambda j: (0, j, 0)),
                  pl.BlockSpec(pmat.shape, lambda j: (0, 0))],
        out_specs=pl.BlockSpec((br, c), lambda j: (j, 0)),
        out_shape=jax.ShapeDtypeStruct((nc_p, c), jnp.float32),
    )(g7, pmat)


def _tc_final(x5, wfc, bfc, n):
    n_p, c = x5.shape

    def body(x_ref, w_ref, b_ref, out_ref):
        rows = lax.broadcasted_iota(jnp.int32, (n_p, 1), 0)
        xm = jnp.where(rows < n, x_ref[...], 0.0)
        s = jnp.sum(xm, axis=0, keepdims=True) * (1.0 / n)
        out_ref[...] = (
            jnp.dot(s, w_ref[...], preferred_element_type=jnp.float32, precision=lax.Precision.HIGHEST)
            + b_ref[0:1, :]
        )

    return pl.pallas_call(
        body,
        out_shape=jax.ShapeDtypeStruct((1, 2), jnp.float32),
    )(x5, wfc, bfc.reshape(1, 2))


def _gather_taps(table, no2, n, n_p):
    c = table.shape[1]
    no_p = jnp.pad(no2, ((0, n_p - n), (0, 0)))
    idx_t = no_p.T
    if c >= 128:
        return _sc_gather(table, idx_t, (7, n_p, c))
    tpr = 128 // c
    s_slabs = -(-7 // tpr)
    idx_t = jnp.pad(idx_t, ((0, s_slabs * tpr - 7), (0, 0)))
    return _sc_gather(table, idx_t, (s_slabs, n_p, 128))


def kernel(neigh_0, neigh_1, neigh_2, neigh_3, neigh_4, neigh_5, x,
           W0, b0, g0, e0, W1, b1, g1, e1, W2, b2, g2, e2, W3, b3, g3, e3,
           W4, b4, g4, e4, W5, b5, g5, e5, W6, b6, g6, e6, W7, b7, g7, e7,
           W8, b8, g8, e8, W9, b9, g9, e9, W10, b10, g10, e10,
           W11, b11, g11, e11, W12, b12, g12, e12, Wfc, bfc):
    neighs = (neigh_0, neigh_1, neigh_2, neigh_3, neigh_4, neigh_5)
    ws = (W0, W1, W2, W3, W4, W5, W6, W7, W8, W9, W10, W11, W12)
    bges = (
        (b0, g0, e0), (b1, g1, e1), (b2, g2, e2), (b3, g3, e3),
        (b4, g4, e4), (b5, g5, e5), (b6, g6, e6), (b7, g7, e7),
        (b8, g8, e8), (b9, g9, e9), (b10, g10, e10), (b11, g11, e11),
        (b12, g12, e12),
    )

    def conv(h, no2, ci, cin, level):
        n, n_p, br = _NS[level], _NPS[level], _BRS[level]
        w = ws[ci]
        cin_p = h.shape[1]
        if cin_p != cin:
            w = jnp.pad(w.reshape(7, cin, -1),
                        ((0, 0), (0, cin_p - cin), (0, 0)))
            w = w.reshape(7 * cin_p, -1)
        if cin_p < 128:
            tpr = 128 // cin_p
            s_slabs = -(-7 // tpr)
            w = jnp.pad(w, ((0, s_slabs * 128 - 7 * cin_p), (0, 0)))
        b, g, e = bges[ci]
        bge = jnp.stack([b, g, e])
        m7 = _gather_taps(h, no2, n, n_p)
        return _tc_conv(m7, w, bge, n, br)

    n0, np0 = _NS[0], _NPS[0]
    h = jnp.pad(x, ((0, np0 - n0), (0, 16 - _CHS[0])))
    no0 = neigh_0.reshape(_NS[0], 7)
    h = conv(h, no0, 0, _CHS[0], 0)
    h = conv(h, no0, 1, _CHS[1], 0)
    h = conv(h, no0, 2, _CHS[1], 0)

    ci = 3
    for l in range(1, 6):
        nc, nc_p, br = _NS[l], _NPS[l], _BRS[l]
        c = _CHS[l]
        nop = neighs[l - 1][: nc * 7].reshape(nc, 7)
        g7 = _gather_taps(h, nop, nc, nc_p)
        pmat = None
        if c < 128:
            s_slabs = -(-7 // (128 // c))
            pmat = jnp.pad(
                jnp.tile(jnp.eye(c, dtype=jnp.float32) * (1.0 / 7.0), (7, 1)),
                ((0, s_slabs * 128 - 7 * c), (0, 0)))
        h = _tc_poolmean(g7, br, pmat)
        no_l = neighs[l].reshape(nc, 7)
        h = conv(h, no_l, ci, _CHS[l], l)
        h = conv(h, no_l, ci + 1, _CHS[l + 1], l)
        ci += 2

    return _tc_final(h, Wfc, bfc, _NS[5])


# speedup vs baseline: 1.0352x; 1.0352x over previous
import functools

import jax
import jax.numpy as jnp
from jax import lax
from jax.experimental import pallas as pl
from jax.experimental.pallas import tpu as pltpu
from jax.experimental.pallas import tpu_sc as plsc

_CHS = [3, 32, 64, 128, 256, 512, 1024]
_NS = [40962, 10242, 2562, 642, 162, 42]
_NPS = [41216, 10752, 2816, 704, 192, 48]
_BRS = [1792, 1536, 704, 176, 192, 48]
_NW = 32


def _sc_gather(table, idx2, out_shape):
    v, c = table.shape
    n_idx_rows, bs = idx2.shape
    s_slabs, _, wout = out_shape
    tpr = wout // c
    buf_mult = 2 if tpr > 1 else 1
    t_max = max(8, (230 * 1024 // (buf_mult * 4 * (wout + tpr))) // 8 * 8)
    flat = s_slabs * bs
    k = max(2, -(-flat // (_NW * t_max)))
    while True:
        nts = max(1, (_NW * k) // s_slabs)
        t = min(t_max, -(-(-(-bs // nts)) // 8) * 8)
        if -(-bs // t) * s_slabs <= _NW * k:
            break
        k += 1
    nts = -(-bs // t)
    nt = s_slabs * nts
    nslots = min(2, k)
    mesh = plsc.VectorSubcoreMesh(core_axis_name="c", subcore_axis_name="s")

    @functools.partial(
        pl.kernel,
        out_type=jax.ShapeDtypeStruct(out_shape, jnp.float32),
        mesh=mesh,
        scratch_types=[
            [pltpu.VMEM((tpr, t), jnp.int32) for _ in range(nslots)],
            [pltpu.VMEM((tpr * t, c), jnp.float32) for _ in range(nslots)],
            [pltpu.VMEM((t, wout), jnp.float32)
             for _ in range(nslots if tpr > 1 else 0)],
            [pltpu.SemaphoreType.DMA for _ in range(3 * nslots)],
        ],
        compiler_params=pltpu.CompilerParams(use_tc_tiling_on_sc=False),
    )
    def gk(table_hbm, idx_hbm, out_hbm, idx_v, rows_g, rows_w, sems):
        wid = lax.axis_index("s") * 2 + lax.axis_index("c")

        def pos(i):
            tile = jnp.minimum(wid + i * _NW, nt - 1)
            slab = tile // nts
            r0 = jnp.minimum((tile % nts) * t, bs - t)
            return slab, r0

        def start_idx(i, slot):
            slab, r0 = pos(i)
            return pltpu.async_copy(
                idx_hbm.at[pl.ds(slab * tpr, tpr), pl.ds(r0, t)],
                idx_v[slot], sems[slot],
            )

        def start_gathers(i, slot):
            hs = []
            for q in range(tpr):
                dst = (rows_g[slot] if tpr == 1
                       else rows_g[slot].at[pl.ds(q * t, t), :])
                hs.append(pltpu.async_copy(
                    table_hbm.at[idx_v[slot].at[q]], dst, sems[nslots + slot]))
            return hs

        def permute(slot):
            if tpr == 1:
                return

            def body(r, carry):
                for q in range(tpr):
                    for ch in range(c // 16):
                        rows_w[slot][r, pl.ds(q * c + ch * 16, 16)] = (
                            rows_g[slot][q * t + r, pl.ds(ch * 16, 16)])
                return carry

            lax.fori_loop(0, t, body, 0)

        def start_wb(i, slot):
            slab, r0 = pos(i)
            src = rows_g[slot] if tpr == 1 else rows_w[slot]
            return pltpu.async_copy(
                src, out_hbm.at[slab, pl.ds(r0, t)],
                sems[2 * nslots + slot],
            )

        idx_h = [None] * k
        g_h = [None] * k
        w_h = [None] * k
        for i in range(nslots):
            idx_h[i] = start_idx(i, i)
        for i in range(k):
            s = i % 2
            idx_h[i].wait()
            if i >= 2:
                w_h[i - 2].wait()
            g_h[i] = start_gathers(i, s)
            if i >= 1:
                for h in g_h[i - 1]:
                    h.wait()
                permute(1 - s)
                w_h[i - 1] = start_wb(i - 1, 1 - s)
                if 2 <= i + 1 < k:
                    idx_h[i + 1] = start_idx(i + 1, 1 - s)
        for h in g_h[k - 1]:
            h.wait()
        permute((k - 1) % 2)
        w_h[k - 1] = start_wb(k - 1, (k - 1) % 2)
        if k >= 2:
            w_h[k - 2].wait()
        w_h[k - 1].wait()

    return gk(table, idx2)


def _bn_lrelu(y, stats_sum, stats_sq, n, g, e):
    m = stats_sum * (1.0 / n)
    var = stats_sq * (1.0 / n) - m * m
    scale = g * lax.rsqrt(var + 1e-5)
    shift = e - m * scale
    yv = y * scale + shift
    return jnp.where(yv >= 0, yv, 0.1 * yv)


def _tc_conv(m7, w, bge, n, br):
    fused = m7.ndim == 2
    n_p = m7.shape[0] if fused else m7.shape[1]
    cout = w.shape[-1]
    nb = n_p // br

    nslab = None if fused else m7.shape[0]
    cw = None if fused else m7.shape[2]

    def matmul(m_ref, w_ref):
        if fused:
            return jnp.dot(m_ref[...], w_ref[...],
                           preferred_element_type=jnp.float32)
        y = jnp.dot(m_ref[0], w_ref[pl.ds(0, cw), :],
                    preferred_element_type=jnp.float32)
        for q in range(1, nslab):
            y += jnp.dot(m_ref[q], w_ref[pl.ds(q * cw, cw), :],
                         preferred_element_type=jnp.float32)
        return y

    if nb == 1:
        def body1(m_ref, w_ref, bge_ref, out_ref):
            y = matmul(m_ref, w_ref) + bge_ref[0:1, :]
            rows = lax.broadcasted_iota(jnp.int32, (n_p, 1), 0)
            ym = jnp.where(rows < n, y, 0.0)
            s1 = jnp.sum(ym, axis=0, keepdims=True)
            s2 = jnp.sum(ym * ym, axis=0, keepdims=True)
            out_ref[...] = _bn_lrelu(y, s1, s2, n, bge_ref[1:2, :],
                                     bge_ref[2:3, :])

        return pl.pallas_call(
            body1,
            out_shape=jax.ShapeDtypeStruct((n_p, cout), jnp.float32),
        )(m7, w, bge)

    if fused:
        m_spec = pl.BlockSpec((br, m7.shape[1]), lambda p, j: ((1 - p) * j, 0))
        w_spec = pl.BlockSpec(w.shape, lambda p, j: (0, 0))
    else:
        m_spec = pl.BlockSpec((nslab, br, cw),
                              lambda p, j: (0, (1 - p) * j, 0))
        w_spec = pl.BlockSpec(w.shape, lambda p, j: (0, 0))

    def body(m_ref, w_ref, bge_ref, out_ref, yacc, stats):
        p = pl.program_id(0)
        j = pl.program_id(1)

        @pl.when(p == 0)
        def _():
            y = matmul(m_ref, w_ref) + bge_ref[0:1, :]
            yacc[pl.ds(j * br, br), :] = y
            rows = j * br + lax.broadcasted_iota(jnp.int32, (br, 1), 0)
            ym = jnp.where(rows < n, y, 0.0)
            s1 = jnp.sum(ym, axis=0, keepdims=True)
            s2 = jnp.sum(ym * ym, axis=0, keepdims=True)

            @pl.when(j == 0)
            def _():
                stats[0:1, :] = s1
                stats[1:2, :] = s2

            @pl.when(j > 0)
            def _():
                stats[0:1, :] += s1
                stats[1:2, :] += s2

        @pl.when(p == 1)
        def _():
            out_ref[...] = _bn_lrelu(
                yacc[pl.ds(j * br, br), :], stats[0:1, :], stats[1:2, :],
                n, bge_ref[1:2, :], bge_ref[2:3, :])

    return pl.pallas_call(
        body,
        grid=(2, nb),
        in_specs=[m_spec, w_spec,
                  pl.BlockSpec((3, cout), lambda p, j: (0, 0))],
        out_specs=pl.BlockSpec((br, cout), lambda p, j: (p * j, 0)),
        out_shape=jax.ShapeDtypeStruct((n_p, cout), jnp.float32),
        scratch_shapes=[
            pltpu.VMEM((n_p, cout), jnp.float32),
            pltpu.VMEM((2, cout), jnp.float32),
        ],
    )(m7, w, bge)


def _tc_poolmean(g7, br, pmat=None):
    nslab, nc_p, cw = g7.shape

    if pmat is None:
        def body2(g_ref, out_ref):
            out_ref[...] = jnp.sum(g_ref[...], axis=0) * (1.0 / 7.0)

        return pl.pallas_call(
            body2,
            grid=(nc_p // br,),
            in_specs=[pl.BlockSpec((nslab, br, cw), lambda j: (0, j, 0))],
            out_specs=pl.BlockSpec((br, cw), lambda j: (j, 0)),
            out_shape=jax.ShapeDtypeStruct((nc_p, cw), jnp.float32),
        )(g7)

    c = pmat.shape[1]

    def body3(g_ref, p_ref, out_ref):
        y = jnp.dot(g_ref[0], p_ref[pl.ds(0, cw), :],
                    preferred_element_type=jnp.float32, precision=lax.Precision.HIGHEST)
        for q in range(1, nslab):
            y += jnp.dot(g_ref[q], p_ref[pl.ds(q * cw, cw), :],
                         preferred_element_type=jnp.float32, precision=lax.Precision.HIGHEST)
        out_ref[...] = y

    return pl.pallas_call(
        body3,
        grid=(nc_p // br,),
        in_specs=[pl.BlockSpec((nslab, br, cw), lambda j: (0, j, 0)),
                  pl.BlockSpec(pmat.shape, lambda j: (0, 0))],
        out_specs=pl.BlockSpec((br, c), lambda j: (j, 0)),
        out_shape=jax.ShapeDtypeStruct((nc_p, c), jnp.float32),
    )(g7, pmat)


def _tc_final(x5, wfc, bfc, n):
    n_p, c = x5.shape

    def body(x_ref, w_ref, b_ref, out_ref):
        rows = lax.broadcasted_iota(jnp.int32, (n_p, 1), 0)
        xm = jnp.where(rows < n, x_ref[...], 0.0)
        s = jnp.sum(xm, axis=0, keepdims=True) * (1.0 / n)
        out_ref[...] = (
            jnp.dot(s, w_ref[...], preferred_element_type=jnp.float32, precision=lax.Precision.HIGHEST)
            + b_ref[0:1, :]
        )

    return pl.pallas_call(
        body,
        out_shape=jax.ShapeDtypeStruct((1, 2), jnp.float32),
    )(x5, wfc, bfc.reshape(1, 2))


def _gather_taps(table, no2, n, n_p):
    c = table.shape[1]
    no_p = jnp.pad(no2, ((0, n_p - n), (0, 0)))
    idx_t = no_p.T
    if c >= 128:
        return _sc_gather(table, idx_t, (7, n_p, c))
    tpr = 128 // c
    s_slabs = -(-7 // tpr)
    idx_t = jnp.pad(idx_t, ((0, s_slabs * tpr - 7), (0, 0)))
    return _sc_gather(table, idx_t, (s_slabs, n_p, 128))


def kernel(neigh_0, neigh_1, neigh_2, neigh_3, neigh_4, neigh_5, x,
           W0, b0, g0, e0, W1, b1, g1, e1, W2, b2, g2, e2, W3, b3, g3, e3,
           W4, b4, g4, e4, W5, b5, g5, e5, W6, b6, g6, e6, W7, b7, g7, e7,
           W8, b8, g8, e8, W9, b9, g9, e9, W10, b10, g10, e10,
           W11, b11, g11, e11, W12, b12, g12, e12, Wfc, bfc):
    neighs = (neigh_0, neigh_1, neigh_2, neigh_3, neigh_4, neigh_5)
    ws = (W0, W1, W2, W3, W4, W5, W6, W7, W8, W9, W10, W11, W12)
    bges = (
        (b0, g0, e0), (b1, g1, e1), (b2, g2, e2), (b3, g3, e3),
        (b4, g4, e4), (b5, g5, e5), (b6, g6, e6), (b7, g7, e7),
        (b8, g8, e8), (b9, g9, e9), (b10, g10, e10), (b11, g11, e11),
        (b12, g12, e12),
    )

    def conv(h, no2, ci, cin, level):
        n, n_p, br = _NS[level], _NPS[level], _BRS[level]
        w = ws[ci]
        cin_p = h.shape[1]
        if cin_p != cin:
            w = jnp.pad(w.reshape(7, cin, -1),
                        ((0, 0), (0, cin_p - cin), (0, 0)))
            w = w.reshape(7 * cin_p, -1)
        if cin_p < 128:
            tpr = 128 // cin_p
            s_slabs = -(-7 // tpr)
            w = jnp.pad(w, ((0, s_slabs * 128 - 7 * cin_p), (0, 0)))
        b, g, e = bges[ci]
        bge = jnp.stack([b, g, e])
        m7 = _gather_taps(h, no2, n, n_p)
        return _tc_conv(m7, w, bge, n, br)

    n0, np0 = _NS[0], _NPS[0]
    h = jnp.pad(x, ((0, np0 - n0), (0, 16 - _CHS[0])))
    no0 = neigh_0.reshape(_NS[0], 7)
    h = conv(h, no0, 0, _CHS[0], 0)
    h = conv(h, no0, 1, _CHS[1], 0)
    h = conv(h, no0, 2, _CHS[1], 0)

    ci = 3
    for l in range(1, 6):
        nc, nc_p, br = _NS[l], _NPS[l], _BRS[l]
        c = _CHS[l]
        nop = neighs[l - 1][: nc * 7].reshape(nc, 7)
        g7 = _gather_taps(h, nop, nc, nc_p)
        pmat = None
        if c < 128:
            s_slabs = -(-7 // (128 // c))
            pmat = jnp.pad(
                jnp.tile(jnp.eye(c, dtype=jnp.float32) * (1.0 / 7.0), (7, 1)),
                ((0, s_slabs * 128 - 7 * c), (0, 0)))
        h = _tc_poolmean(g7, br, pmat)
        no_l = neighs[l].reshape(nc, 7)
        h = conv(h, no_l, ci, _CHS[l], l)
        h = conv(h, no_l, ci + 1, _CHS[l + 1], l)
        ci += 2

    return _tc_final(h, Wfc, bfc, _NS[5])


# speedup vs baseline: 1.7411x; 1.6820x over previous
import functools

import jax
import jax.numpy as jnp
from jax import lax
from jax.experimental import pallas as pl
from jax.experimental.pallas import tpu as pltpu
from jax.experimental.pallas import tpu_sc as plsc

_CHS = [3, 32, 64, 128, 256, 512, 1024]
_NS = [40962, 10242, 2562, 642, 162, 42]
_NPS = [41216, 10752, 2816, 704, 192, 48]
_BRS = [1792, 1536, 704, 176, 192, 48]
_NW = 32


def _sc_gather(table, idx2, out_shape):
    v, c = table.shape
    n_idx_rows, bs = idx2.shape
    s_slabs, _, wout = out_shape
    tpr = wout // c
    buf_mult = 2 if tpr > 1 else 1
    t_max = max(8, (230 * 1024 // (buf_mult * 4 * (wout + tpr))) // 8 * 8)
    flat = s_slabs * bs
    k = max(2, -(-flat // (_NW * t_max)))
    while True:
        nts = max(1, (_NW * k) // s_slabs)
        t = min(t_max, -(-(-(-bs // nts)) // 8) * 8)
        if -(-bs // t) * s_slabs <= _NW * k:
            break
        k += 1
    nts = -(-bs // t)
    nt = s_slabs * nts
    nslots = min(2, k)
    mesh = plsc.VectorSubcoreMesh(core_axis_name="c", subcore_axis_name="s")

    @functools.partial(
        pl.kernel,
        out_type=jax.ShapeDtypeStruct(out_shape, jnp.float32),
        mesh=mesh,
        scratch_types=[
            [pltpu.VMEM((tpr, t), jnp.int32) for _ in range(nslots)],
            [pltpu.VMEM((tpr * t, c), jnp.float32) for _ in range(nslots)],
            [pltpu.VMEM((t, wout), jnp.float32)
             for _ in range(nslots if tpr > 1 else 0)],
            [pltpu.SemaphoreType.DMA for _ in range(3 * nslots)],
        ],
        compiler_params=pltpu.CompilerParams(use_tc_tiling_on_sc=False),
    )
    def gk(table_hbm, idx_hbm, out_hbm, idx_v, rows_g, rows_w, sems):
        wid = lax.axis_index("s") * 2 + lax.axis_index("c")

        def pos(i):
            tile = jnp.minimum(wid + i * _NW, nt - 1)
            slab = tile // nts
            r0 = jnp.minimum((tile % nts) * t, bs - t)
            return slab, r0

        def start_idx(i, slot):
            slab, r0 = pos(i)
            return pltpu.async_copy(
                idx_hbm.at[pl.ds(slab * tpr, tpr), pl.ds(r0, t)],
                idx_v[slot], sems[slot],
            )

        def start_gathers(i, slot):
            hs = []
            for q in range(tpr):
                dst = (rows_g[slot] if tpr == 1
                       else rows_g[slot].at[pl.ds(q * t, t), :])
                hs.append(pltpu.async_copy(
                    table_hbm.at[idx_v[slot].at[q]], dst, sems[nslots + slot]))
            return hs

        def permute(slot):
            if tpr == 1:
                return

            def body(r, carry):
                for q in range(tpr):
                    for ch in range(c // 16):
                        rows_w[slot][r, pl.ds(q * c + ch * 16, 16)] = (
                            rows_g[slot][q * t + r, pl.ds(ch * 16, 16)])
                return carry

            lax.fori_loop(0, t, body, 0)

        def start_wb(i, slot):
            slab, r0 = pos(i)
            src = rows_g[slot] if tpr == 1 else rows_w[slot]
            return pltpu.async_copy(
                src, out_hbm.at[slab, pl.ds(r0, t)],
                sems[2 * nslots + slot],
            )

        idx_h = [None] * k
        g_h = [None] * k
        w_h = [None] * k
        for i in range(nslots):
            idx_h[i] = start_idx(i, i)
        for i in range(k):
            s = i % 2
            idx_h[i].wait()
            if i >= 2:
                w_h[i - 2].wait()
            g_h[i] = start_gathers(i, s)
            if i >= 1:
                for h in g_h[i - 1]:
                    h.wait()
                permute(1 - s)
                w_h[i - 1] = start_wb(i - 1, 1 - s)
                if 2 <= i + 1 < k:
                    idx_h[i + 1] = start_idx(i + 1, 1 - s)
        for h in g_h[k - 1]:
            h.wait()
        permute((k - 1) % 2)
        w_h[k - 1] = start_wb(k - 1, (k - 1) % 2)
        if k >= 2:
            w_h[k - 2].wait()
        w_h[k - 1].wait()

    return gk(table, idx2)


def _bn_lrelu(y, stats_sum, stats_sq, n, g, e):
    m = stats_sum * (1.0 / n)
    var = stats_sq * (1.0 / n) - m * m
    scale = g * lax.rsqrt(var + 1e-5)
    shift = e - m * scale
    yv = y * scale + shift
    return jnp.where(yv >= 0, yv, 0.1 * yv)


def _tc_conv(m7, w, bge, n, br):
    fused = m7.ndim == 2
    n_p = m7.shape[0] if fused else m7.shape[1]
    cout = w.shape[-1]
    nb = n_p // br

    nslab = None if fused else m7.shape[0]
    cw = None if fused else m7.shape[2]

    def matmul(m_ref, w_ref):
        if fused:
            return jnp.dot(m_ref[...], w_ref[...],
                           preferred_element_type=jnp.float32)
        y = jnp.dot(m_ref[0], w_ref[pl.ds(0, cw), :],
                    preferred_element_type=jnp.float32)
        for q in range(1, nslab):
            y += jnp.dot(m_ref[q], w_ref[pl.ds(q * cw, cw), :],
                         preferred_element_type=jnp.float32)
        return y

    if nb == 1:
        def body1(m_ref, w_ref, bge_ref, out_ref):
            y = matmul(m_ref, w_ref) + bge_ref[0:1, :]
            rows = lax.broadcasted_iota(jnp.int32, (n_p, 1), 0)
            ym = jnp.where(rows < n, y, 0.0)
            s1 = jnp.sum(ym, axis=0, keepdims=True)
            s2 = jnp.sum(ym * ym, axis=0, keepdims=True)
            out_ref[...] = _bn_lrelu(y, s1, s2, n, bge_ref[1:2, :],
                                     bge_ref[2:3, :])

        return pl.pallas_call(
            body1,
            out_shape=jax.ShapeDtypeStruct((n_p, cout), jnp.float32),
        )(m7, w, bge)

    if fused:
        m_spec = pl.BlockSpec((br, m7.shape[1]), lambda p, j: ((1 - p) * j, 0))
        w_spec = pl.BlockSpec(w.shape, lambda p, j: (0, 0))
    else:
        m_spec = pl.BlockSpec((nslab, br, cw),
                              lambda p, j: (0, (1 - p) * j, 0))
        w_spec = pl.BlockSpec(w.shape, lambda p, j: (0, 0))

    def body(m_ref, w_ref, bge_ref, out_ref, yacc, stats):
        p = pl.program_id(0)
        j = pl.program_id(1)

        @pl.when(p == 0)
        def _():
            y = matmul(m_ref, w_ref) + bge_ref[0:1, :]
            yacc[pl.ds(j * br, br), :] = y
            rows = j * br + lax.broadcasted_iota(jnp.int32, (br, 1), 0)
            ym = jnp.where(rows < n, y, 0.0)
            s1 = jnp.sum(ym, axis=0, keepdims=True)
            s2 = jnp.sum(ym * ym, axis=0, keepdims=True)

            @pl.when(j == 0)
            def _():
                stats[0:1, :] = s1
                stats[1:2, :] = s2

            @pl.when(j > 0)
            def _():
                stats[0:1, :] += s1
                stats[1:2, :] += s2

        @pl.when(p == 1)
        def _():
            out_ref[...] = _bn_lrelu(
                yacc[pl.ds(j * br, br), :], stats[0:1, :], stats[1:2, :],
                n, bge_ref[1:2, :], bge_ref[2:3, :])

    return pl.pallas_call(
        body,
        grid=(2, nb),
        in_specs=[m_spec, w_spec,
                  pl.BlockSpec((3, cout), lambda p, j: (0, 0))],
        out_specs=pl.BlockSpec((br, cout), lambda p, j: (p * j, 0)),
        out_shape=jax.ShapeDtypeStruct((n_p, cout), jnp.float32),
        scratch_shapes=[
            pltpu.VMEM((n_p, cout), jnp.float32),
            pltpu.VMEM((2, cout), jnp.float32),
        ],
    )(m7, w, bge)


def _tc_poolmean(g7, br, pmat=None):
    nslab, nc_p, cw = g7.shape

    if pmat is None:
        def body2(g_ref, out_ref):
            out_ref[...] = jnp.sum(g_ref[...], axis=0) * (1.0 / 7.0)

        return pl.pallas_call(
            body2,
            grid=(nc_p // br,),
            in_specs=[pl.BlockSpec((nslab, br, cw), lambda j: (0, j, 0))],
            out_specs=pl.BlockSpec((br, cw), lambda j: (j, 0)),
            out_shape=jax.ShapeDtypeStruct((nc_p, cw), jnp.float32),
        )(g7)

    c = pmat.shape[1]

    def body3(g_ref, p_ref, out_ref):
        y = jnp.dot(g_ref[0], p_ref[pl.ds(0, cw), :],
                    preferred_element_type=jnp.float32, precision=lax.Precision.HIGHEST)
        for q in range(1, nslab):
            y += jnp.dot(g_ref[q], p_ref[pl.ds(q * cw, cw), :],
                         preferred_element_type=jnp.float32, precision=lax.Precision.HIGHEST)
        out_ref[...] = y

    return pl.pallas_call(
        body3,
        grid=(nc_p // br,),
        in_specs=[pl.BlockSpec((nslab, br, cw), lambda j: (0, j, 0)),
                  pl.BlockSpec(pmat.shape, lambda j: (0, 0))],
        out_specs=pl.BlockSpec((br, c), lambda j: (j, 0)),
        out_shape=jax.ShapeDtypeStruct((nc_p, c), jnp.float32),
    )(g7, pmat)


def _tc_final(x5, wfc, bfc, n):
    n_p, c = x5.shape

    def body(x_ref, w_ref, b_ref, out_ref):
        rows = lax.broadcasted_iota(jnp.int32, (n_p, 1), 0)
        xm = jnp.where(rows < n, x_ref[...], 0.0)
        s = jnp.sum(xm, axis=0, keepdims=True) * (1.0 / n)
        out_ref[...] = (
            jnp.dot(s, w_ref[...], preferred_element_type=jnp.float32, precision=lax.Precision.HIGHEST)
            + b_ref[0:1, :]
        )

    return pl.pallas_call(
        body,
        out_shape=jax.ShapeDtypeStruct((1, 2), jnp.float32),
    )(x5, wfc, bfc.reshape(1, 2))


def _gather_taps(table, no2, n, n_p):
    c = table.shape[1]
    no_p = jnp.pad(no2, ((0, n_p - n), (0, 0)))
    return _sc_gather(table, no_p.T, (7, n_p, c))


def kernel(neigh_0, neigh_1, neigh_2, neigh_3, neigh_4, neigh_5, x,
           W0, b0, g0, e0, W1, b1, g1, e1, W2, b2, g2, e2, W3, b3, g3, e3,
           W4, b4, g4, e4, W5, b5, g5, e5, W6, b6, g6, e6, W7, b7, g7, e7,
           W8, b8, g8, e8, W9, b9, g9, e9, W10, b10, g10, e10,
           W11, b11, g11, e11, W12, b12, g12, e12, Wfc, bfc):
    neighs = (neigh_0, neigh_1, neigh_2, neigh_3, neigh_4, neigh_5)
    ws = (W0, W1, W2, W3, W4, W5, W6, W7, W8, W9, W10, W11, W12)
    bges = (
        (b0, g0, e0), (b1, g1, e1), (b2, g2, e2), (b3, g3, e3),
        (b4, g4, e4), (b5, g5, e5), (b6, g6, e6), (b7, g7, e7),
        (b8, g8, e8), (b9, g9, e9), (b10, g10, e10), (b11, g11, e11),
        (b12, g12, e12),
    )

    def conv(h, no2, ci, cin, level):
        n, n_p, br = _NS[level], _NPS[level], _BRS[level]
        w = ws[ci]
        cin_p = h.shape[1]
        if cin_p != cin:
            w = jnp.pad(w.reshape(7, cin, -1),
                        ((0, 0), (0, cin_p - cin), (0, 0)))
            w = w.reshape(7 * cin_p, -1)
        b, g, e = bges[ci]
        bge = jnp.stack([b, g, e])
        m7 = _gather_taps(h, no2, n, n_p)
        return _tc_conv(m7, w, bge, n, br)

    n0, np0 = _NS[0], _NPS[0]
    h = jnp.pad(x, ((0, np0 - n0), (0, 16 - _CHS[0])))
    no0 = neigh_0.reshape(_NS[0], 7)
    h = conv(h, no0, 0, _CHS[0], 0)
    h = conv(h, no0, 1, _CHS[1], 0)
    h = conv(h, no0, 2, _CHS[1], 0)

    ci = 3
    for l in range(1, 6):
        nc, nc_p, br = _NS[l], _NPS[l], _BRS[l]
        c = _CHS[l]
        nop = neighs[l - 1][: nc * 7].reshape(nc, 7)
        g7 = _gather_taps(h, nop, nc, nc_p)
        h = _tc_poolmean(g7, br)
        no_l = neighs[l].reshape(nc, 7)
        h = conv(h, no_l, ci, _CHS[l], l)
        h = conv(h, no_l, ci + 1, _CHS[l + 1], l)
        ci += 2

    return _tc_final(h, Wfc, bfc, _NS[5])
